# baseline (speedup 1.0000x reference)
import functools

import jax
import jax.numpy as jnp
from jax import lax
from jax.experimental import pallas as pl
from jax.experimental.pallas import tpu as pltpu
from jax.experimental.pallas import tpu_sc as plsc

N = 10000
E = 320000
F_IN = 128
HID = 128
C = 7
C64 = 64

NC = 2
NS = 16
NW = NC * NS
CHUNK = 80
NP = 10240
RPT = NP // NS
NPP = NP // 2
HH = HID // 2
SPW = NP // NW
NCHUNK = E // NW // CHUNK + SPW // CHUNK + 1
GRP = 26
NBUF = 4

_MESH = plsc.VectorSubcoreMesh(core_axis_name="c", subcore_axis_name="s")
_SC_PARAMS = pltpu.CompilerParams(use_tc_tiling_on_sc=False)


def _edge_loop(g_hbm, src_v, dst_v, rows_v, acc_sh, gsem, ssem):
  def group(i, _):
    base = i * GRP
    gd, sd = {}, {}
    for k in range(2):
      gd[k] = pltpu.async_copy(g_hbm.at[src_v.at[base + k]],
                               rows_v.at[k], gsem)
    for k in range(GRP):
      if k >= 2:
        sd[k - 2].wait()
      if k + 2 < GRP:
        gd[k + 2] = pltpu.async_copy(g_hbm.at[src_v.at[base + k + 2]],
                                     rows_v.at[(k + 2) % NBUF], gsem)
      gd[k].wait()
      sd[k] = pltpu.async_copy(rows_v.at[k % NBUF],
                               acc_sh.at[dst_v.at[base + k]], ssem, add=True)
    sd[GRP - 2].wait()
    sd[GRP - 1].wait()
    return 0

  lax.fori_loop(0, NCHUNK // GRP, group, 0)


@functools.partial(
    pl.kernel,
    out_type=jax.ShapeDtypeStruct((NC, 2, NP, HH), jnp.float32),
    mesh=_MESH,
    compiler_params=_SC_PARAMS,
    scratch_types=[
        pltpu.VMEM((NCHUNK, CHUNK), jnp.int32),
        pltpu.VMEM((NCHUNK, CHUNK), jnp.int32),
        pltpu.VMEM((NCHUNK, CHUNK), jnp.int32),
        pltpu.VMEM((NBUF, CHUNK, HH), jnp.float32),
        pltpu.VMEM_SHARED((NP, HH), jnp.float32),
        pltpu.SemaphoreType.DMA,
        pltpu.SemaphoreType.DMA,
    ],
)
def _edge_pass_wide(g2n_hbm, srca_hbm, srcb_hbm, dst_hbm, zeros_hbm, out_hbm,
                    srca_v, srcb_v, dst_v, rows_v, acc_sh, gsem, ssem):
  cid = lax.axis_index("c")
  sid = lax.axis_index("s")
  wid = cid * NS + sid
  stripe = pl.ds(sid * RPT, RPT)
  pltpu.sync_copy(srca_hbm.at[wid], srca_v)
  pltpu.sync_copy(srcb_hbm.at[wid], srcb_v)
  pltpu.sync_copy(dst_hbm.at[wid], dst_v)
  pltpu.sync_copy(zeros_hbm.at[stripe], acc_sh.at[stripe])
  plsc.subcore_barrier()
  for h, src_v in enumerate((srca_v, srcb_v)):
    _edge_loop(g2n_hbm, src_v, dst_v, rows_v, acc_sh, gsem, ssem)
    plsc.subcore_barrier()
    pltpu.sync_copy(acc_sh.at[stripe], out_hbm.at[cid, h, stripe])
    if h == 0:
      pltpu.sync_copy(zeros_hbm.at[stripe], acc_sh.at[stripe])
      plsc.subcore_barrier()


@functools.partial(
    pl.kernel,
    out_type=jax.ShapeDtypeStruct((NC, NP, C64), jnp.float32),
    mesh=_MESH,
    compiler_params=_SC_PARAMS,
    scratch_types=[
        pltpu.VMEM((NCHUNK, CHUNK), jnp.int32),
        pltpu.VMEM((NCHUNK, CHUNK), jnp.int32),
        pltpu.VMEM((NBUF, CHUNK, C64), jnp.float32),
        pltpu.VMEM_SHARED((NP, C64), jnp.float32),
        pltpu.SemaphoreType.DMA,
        pltpu.SemaphoreType.DMA,
    ],
)
def _edge_pass_narrow(g_hbm, src_hbm, dst_hbm, zeros_hbm, out_hbm,
                      src_v, dst_v, rows_v, acc_sh, gsem, ssem):
  cid = lax.axis_index("c")
  sid = lax.axis_index("s")
  wid = cid * NS + sid
  stripe = pl.ds(sid * RPT, RPT)
  pltpu.sync_copy(src_hbm.at[wid], src_v)
  pltpu.sync_copy(dst_hbm.at[wid], dst_v)
  pltpu.sync_copy(zeros_hbm.at[stripe], acc_sh.at[stripe])
  plsc.subcore_barrier()
  _edge_loop(g_hbm, src_v, dst_v, rows_v, acc_sh, gsem, ssem)
  plsc.subcore_barrier()
  pltpu.sync_copy(acc_sh.at[stripe], out_hbm.at[cid, stripe])


@functools.partial(
    pl.kernel,
    out_type=jax.ShapeDtypeStruct((NC, NP, 16), jnp.float32),
    mesh=_MESH,
    compiler_params=_SC_PARAMS,
    scratch_types=[
        pltpu.VMEM((NCHUNK, CHUNK), jnp.int32),
        pltpu.VMEM((CHUNK, 16), jnp.float32),
        pltpu.VMEM_SHARED((NP, 16), jnp.float32),
        pltpu.SemaphoreType.DMA,
    ],
)
def _deg_pass(dst_hbm, ones_hbm, zeros_hbm, out_hbm,
              dst_v, ones_v, acc_sh, dsem):
  cid = lax.axis_index("c")
  sid = lax.axis_index("s")
  wid = cid * NS + sid
  stripe = pl.ds(sid * RPT, RPT)
  pltpu.sync_copy(zeros_hbm.at[stripe], acc_sh.at[stripe])
  pltpu.sync_copy(dst_hbm.at[wid], dst_v)
  pltpu.sync_copy(ones_hbm, ones_v)
  plsc.subcore_barrier()

  def group(i, _):
    sd = {}
    for k in range(GRP):
      if k >= 3:
        sd[k - 3].wait()
      sd[k] = pltpu.async_copy(ones_v, acc_sh.at[dst_v.at[i * GRP + k]],
                               dsem, add=True)
    for k in range(GRP - 3, GRP):
      sd[k].wait()
    return 0

  lax.fori_loop(0, NCHUNK // GRP, group, 0)
  plsc.subcore_barrier()
  pltpu.sync_copy(acc_sh.at[stripe], out_hbm.at[cid, stripe])


def _tc_h_body(x_ref, w1_ref, h_ref):
  h_ref[0:N, :] = jnp.dot(x_ref[...], w1_ref[...],
                          preferred_element_type=jnp.float32)


def _tc_g1_body(h_ref, dt_ref, g1_ref):
  dis = lax.rsqrt(dt_ref[:, 0:1] + dt_ref[:, 1:2])
  g1_ref[...] = h_ref[...] * dis
  g1_ref[N:NP, :] = jnp.zeros((NP - N, HID), jnp.float32)


def _tc2_body(ppv_ref, dt2_ref, b1_ref, w2_ref, g2p_ref):
  dis_e = lax.rsqrt(dt2_ref[:, 0:1] + dt2_ref[:, 1:2])
  dis_o = lax.rsqrt(dt2_ref[:, 2:3] + dt2_ref[:, 3:4])
  psum_a = ppv_ref[0, 0] + ppv_ref[1, 0]
  psum_b = ppv_ref[0, 1] + ppv_ref[1, 1]
  agg_e = dis_e * jnp.concatenate(
      [psum_a[:, 0:HH], psum_b[:, 0:HH]], axis=1) + b1_ref[...]
  agg_o = dis_o * jnp.concatenate(
      [psum_a[:, HH:], psum_b[:, HH:]], axis=1) + b1_ref[...]
  h1p = jnp.concatenate(
      [jnp.maximum(agg_e, 0.0), jnp.maximum(agg_o, 0.0)], axis=1)
  h2p = jnp.dot(h1p, w2_ref[...], preferred_element_type=jnp.float32)
  disp = jnp.concatenate([jnp.broadcast_to(dis_e, (NPP, C64)),
                          jnp.broadcast_to(dis_o, (NPP, C64))], axis=1)
  g2p_ref[...] = h2p * disp
  g2p_ref[N // 2:NPP, :] = jnp.zeros((NPP - N // 2, 128), jnp.float32)


def _tc3_body(qqv_ref, dt2_ref, b2_ref, outp_ref):
  dis_e = lax.rsqrt(dt2_ref[:, 0:1] + dt2_ref[:, 1:2])
  dis_o = lax.rsqrt(dt2_ref[:, 2:3] + dt2_ref[:, 3:4])
  disp = jnp.concatenate([jnp.broadcast_to(dis_e, (NPP, C64)),
                          jnp.broadcast_to(dis_o, (NPP, C64))], axis=1)
  outp_ref[...] = disp * (qqv_ref[0] + qqv_ref[1]) + b2_ref[...]


def _one_block(body, out_shape):
  return pl.pallas_call(
      body,
      out_shape=jax.ShapeDtypeStruct(out_shape, jnp.float32),
  )


def kernel(x, edge_index, W1, b1, W2, b2):
  i32 = jnp.int32
  selfn = jnp.arange(NP, dtype=i32).reshape(NW, SPW // CHUNK, CHUNK)
  padn = jnp.broadcast_to(jnp.arange(NP - CHUNK, NP, dtype=i32),
                          (NW, 1, CHUNK))
  src = edge_index[0].reshape(NW, E // NW // CHUNK, CHUNK)
  dstr = edge_index[1].reshape(NW, E // NW // CHUNK, CHUNK)
  srcf = jnp.concatenate([src, selfn, padn], axis=1)
  dstf = jnp.concatenate([dstr, selfn, padn], axis=1)
  srca = 2 * srcf
  srcb = srca + 1
  zeros64 = jnp.zeros((NP, HH), jnp.float32)
  zeros16 = jnp.zeros((NP, 16), jnp.float32)
  ones16 = jnp.ones((CHUNK, 16), jnp.float32)
  w2p = jnp.pad(W2, ((0, 0), (0, C64 - C)))
  w2pp = jnp.kron(jnp.eye(2, dtype=jnp.float32), w2p)
  b1r = b1.reshape(1, HID)
  b2p = jnp.tile(jnp.pad(b2, (0, C64 - C)), 2).reshape(1, 128)

  degp = _deg_pass(dstf, ones16, zeros16)
  degt = jnp.stack([degp[0, :, 0], degp[1, :, 0]], axis=1)
  d2 = degp[:, :, 0].reshape(NC, NPP, 2)
  degt2 = jnp.stack([d2[0, :, 0], d2[1, :, 0], d2[0, :, 1], d2[1, :, 1]],
                    axis=1)

  h = _one_block(_tc_h_body, (NP, HID))(x, W1)
  g1 = _one_block(_tc_g1_body, (NP, HID))(h, degt)

  g2n = g1.reshape(2 * NP, HH)
  pp1 = _edge_pass_wide(g2n, srca, srcb, dstf, zeros64)
  ppv = pp1.reshape(NC, 2, NPP, 128)

  g2p = _one_block(_tc2_body, (NPP, 128))(ppv, degt2, b1r, w2pp)

  g2view = g2p.reshape(NP, C64)
  pp2 = _edge_pass_narrow(g2view, srcf, dstf, zeros64)
  qqv = pp2.reshape(NC, NPP, 128)

  outp = _one_block(_tc3_body, (NPP, 128))(qqv, degt2, b2p)
  return outp.reshape(NP, C64)[:N, :C]


# speedup vs baseline: 1.0140x; 1.0140x over previous
import functools

import jax
import jax.numpy as jnp
from jax import lax
from jax.experimental import pallas as pl
from jax.experimental.pallas import tpu as pltpu
from jax.experimental.pallas import tpu_sc as plsc

N = 10000
E = 320000
F_IN = 128
HID = 128
C = 7
C64 = 64
CP16 = 16

NC = 2
NS = 16
NW = NC * NS
CHUNK = 80
NP = 10240
RPT = NP // NS
NPP = NP // 2
HH = HID // 2
SPW = NP // NW
NCHUNK = E // NW // CHUNK + SPW // CHUNK + 1
GRP = 26
NBUF = 4

_MESH = plsc.VectorSubcoreMesh(core_axis_name="c", subcore_axis_name="s")
_SC_PARAMS = pltpu.CompilerParams(use_tc_tiling_on_sc=False)


def _edge_loop(g_hbm, src_v, dst_v, rows_v, acc_sh, gsem, ssem):
  def group(i, _):
    base = i * GRP
    gd, sd = {}, {}
    for k in range(2):
      gd[k] = pltpu.async_copy(g_hbm.at[src_v.at[base + k]],
                               rows_v.at[k], gsem)
    for k in range(GRP):
      if k >= 2:
        sd[k - 2].wait()
      if k + 2 < GRP:
        gd[k + 2] = pltpu.async_copy(g_hbm.at[src_v.at[base + k + 2]],
                                     rows_v.at[(k + 2) % NBUF], gsem)
      gd[k].wait()
      sd[k] = pltpu.async_copy(rows_v.at[k % NBUF],
                               acc_sh.at[dst_v.at[base + k]], ssem, add=True)
    sd[GRP - 2].wait()
    sd[GRP - 1].wait()
    return 0

  lax.fori_loop(0, NCHUNK // GRP, group, 0)


@functools.partial(
    pl.kernel,
    out_type=jax.ShapeDtypeStruct((NC, 2, NP, HH), jnp.float32),
    mesh=_MESH,
    compiler_params=_SC_PARAMS,
    scratch_types=[
        pltpu.VMEM((NCHUNK, CHUNK), jnp.int32),
        pltpu.VMEM((NCHUNK, CHUNK), jnp.int32),
        pltpu.VMEM((NCHUNK, CHUNK), jnp.int32),
        pltpu.VMEM((NBUF, CHUNK, HH), jnp.float32),
        pltpu.VMEM_SHARED((NP, HH), jnp.float32),
        pltpu.SemaphoreType.DMA,
        pltpu.SemaphoreType.DMA,
    ],
)
def _edge_pass_wide(g2n_hbm, srca_hbm, srcb_hbm, dst_hbm, zeros_hbm, out_hbm,
                    srca_v, srcb_v, dst_v, rows_v, acc_sh, gsem, ssem):
  cid = lax.axis_index("c")
  sid = lax.axis_index("s")
  wid = cid * NS + sid
  stripe = pl.ds(sid * RPT, RPT)
  pltpu.sync_copy(srca_hbm.at[wid], srca_v)
  pltpu.sync_copy(srcb_hbm.at[wid], srcb_v)
  pltpu.sync_copy(dst_hbm.at[wid], dst_v)
  pltpu.sync_copy(zeros_hbm.at[stripe], acc_sh.at[stripe])
  plsc.subcore_barrier()
  for h, src_v in enumerate((srca_v, srcb_v)):
    _edge_loop(g2n_hbm, src_v, dst_v, rows_v, acc_sh, gsem, ssem)
    plsc.subcore_barrier()
    pltpu.sync_copy(acc_sh.at[stripe], out_hbm.at[cid, h, stripe])
    if h == 0:
      pltpu.sync_copy(zeros_hbm.at[stripe], acc_sh.at[stripe])
      plsc.subcore_barrier()


@functools.partial(
    pl.kernel,
    out_type=jax.ShapeDtypeStruct((NC, NP, CP16), jnp.float32),
    mesh=_MESH,
    compiler_params=_SC_PARAMS,
    scratch_types=[
        pltpu.VMEM((NCHUNK, CHUNK), jnp.int32),
        pltpu.VMEM((NCHUNK, CHUNK), jnp.int32),
        pltpu.VMEM((NBUF, CHUNK, CP16), jnp.float32),
        pltpu.VMEM_SHARED((NP, CP16), jnp.float32),
        pltpu.SemaphoreType.DMA,
        pltpu.SemaphoreType.DMA,
    ],
)
def _edge_pass_narrow(g_hbm, src_hbm, dst_hbm, zeros_hbm, out_hbm,
                      src_v, dst_v, rows_v, acc_sh, gsem, ssem):
  cid = lax.axis_index("c")
  sid = lax.axis_index("s")
  wid = cid * NS + sid
  stripe = pl.ds(sid * RPT, RPT)
  pltpu.sync_copy(src_hbm.at[wid], src_v)
  pltpu.sync_copy(dst_hbm.at[wid], dst_v)
  pltpu.sync_copy(zeros_hbm.at[stripe], acc_sh.at[stripe])
  plsc.subcore_barrier()
  _edge_loop(g_hbm, src_v, dst_v, rows_v, acc_sh, gsem, ssem)
  plsc.subcore_barrier()
  pltpu.sync_copy(acc_sh.at[stripe], out_hbm.at[cid, stripe])


@functools.partial(
    pl.kernel,
    out_type=jax.ShapeDtypeStruct((NC, NP, 16), jnp.float32),
    mesh=_MESH,
    compiler_params=_SC_PARAMS,
    scratch_types=[
        pltpu.VMEM((NCHUNK, CHUNK), jnp.int32),
        pltpu.VMEM((CHUNK, 16), jnp.float32),
        pltpu.VMEM_SHARED((NP, 16), jnp.float32),
        pltpu.SemaphoreType.DMA,
    ],
)
def _deg_pass(dst_hbm, ones_hbm, zeros_hbm, out_hbm,
              dst_v, ones_v, acc_sh, dsem):
  cid = lax.axis_index("c")
  sid = lax.axis_index("s")
  wid = cid * NS + sid
  stripe = pl.ds(sid * RPT, RPT)
  pltpu.sync_copy(zeros_hbm.at[stripe], acc_sh.at[stripe])
  pltpu.sync_copy(dst_hbm.at[wid], dst_v)
  pltpu.sync_copy(ones_hbm, ones_v)
  plsc.subcore_barrier()

  def group(i, _):
    sd = {}
    for k in range(GRP):
      if k >= 3:
        sd[k - 3].wait()
      sd[k] = pltpu.async_copy(ones_v, acc_sh.at[dst_v.at[i * GRP + k]],
                               dsem, add=True)
    for k in range(GRP - 3, GRP):
      sd[k].wait()
    return 0

  lax.fori_loop(0, NCHUNK // GRP, group, 0)
  plsc.subcore_barrier()
  pltpu.sync_copy(acc_sh.at[stripe], out_hbm.at[cid, stripe])


def _tc_h_body(x_ref, w1_ref, h_ref):
  h_ref[0:N, :] = jnp.dot(x_ref[...], w1_ref[...],
                          preferred_element_type=jnp.float32)


def _tc_g1_body(h_ref, dt_ref, g1_ref):
  dis = lax.rsqrt(dt_ref[:, 0:1] + dt_ref[:, 1:2])
  g1_ref[...] = h_ref[...] * dis
  g1_ref[N:NP, :] = jnp.zeros((NP - N, HID), jnp.float32)


def _tc2_body(ppv_ref, dt2_ref, b1_ref, w2_ref, g2p_ref):
  dis_e = lax.rsqrt(dt2_ref[:, 0:1] + dt2_ref[:, 1:2])
  dis_o = lax.rsqrt(dt2_ref[:, 2:3] + dt2_ref[:, 3:4])
  psum_a = ppv_ref[0, 0] + ppv_ref[1, 0]
  psum_b = ppv_ref[0, 1] + ppv_ref[1, 1]
  agg_e = dis_e * jnp.concatenate(
      [psum_a[:, 0:HH], psum_b[:, 0:HH]], axis=1) + b1_ref[...]
  agg_o = dis_o * jnp.concatenate(
      [psum_a[:, HH:], psum_b[:, HH:]], axis=1) + b1_ref[...]
  h1p = jnp.concatenate(
      [jnp.maximum(agg_e, 0.0), jnp.maximum(agg_o, 0.0)], axis=1)
  h2p = jnp.dot(h1p, w2_ref[...], preferred_element_type=jnp.float32)
  disp = jnp.concatenate([jnp.broadcast_to(dis_e, (NPP, CP16)),
                          jnp.broadcast_to(dis_o, (NPP, CP16))], axis=1)
  g2p_ref[...] = h2p * disp
  g2p_ref[N // 2:NPP, :] = jnp.zeros((NPP - N // 2, 2 * CP16), jnp.float32)


NP8 = NP // 8


def _tc3_body(qqv_ref, dt8_ref, b2_ref, outp_ref):
  disrow = jnp.concatenate(
      [jnp.broadcast_to(
          lax.rsqrt(dt8_ref[:, 2 * k:2 * k + 1] + dt8_ref[:, 2 * k + 1:2 * k + 2]),
          (NP8, CP16)) for k in range(8)], axis=1)
  outp_ref[...] = disrow * (qqv_ref[0] + qqv_ref[1]) + b2_ref[...]


def _one_block(body, out_shape):
  return pl.pallas_call(
      body,
      out_shape=jax.ShapeDtypeStruct(out_shape, jnp.float32),
  )


def kernel(x, edge_index, W1, b1, W2, b2):
  i32 = jnp.int32
  selfn = jnp.arange(NP, dtype=i32).reshape(NW, SPW // CHUNK, CHUNK)
  padn = jnp.broadcast_to(jnp.arange(NP - CHUNK, NP, dtype=i32),
                          (NW, 1, CHUNK))
  src = edge_index[0].reshape(NW, E // NW // CHUNK, CHUNK)
  dstr = edge_index[1].reshape(NW, E // NW // CHUNK, CHUNK)
  srcf = jnp.concatenate([src, selfn, padn], axis=1)
  dstf = jnp.concatenate([dstr, selfn, padn], axis=1)
  srca = 2 * srcf
  srcb = srca + 1
  zeros64 = jnp.zeros((NP, HH), jnp.float32)
  zeros16 = jnp.zeros((NP, 16), jnp.float32)
  ones16 = jnp.ones((CHUNK, 16), jnp.float32)
  w2p = jnp.pad(W2, ((0, 0), (0, CP16 - C)))
  w2pp = jnp.kron(jnp.eye(2, dtype=jnp.float32), w2p)
  b1r = b1.reshape(1, HID)
  b2q = jnp.tile(jnp.pad(b2, (0, CP16 - C)), 8).reshape(1, 128)

  degp = _deg_pass(dstf, ones16, zeros16)
  degt = jnp.stack([degp[0, :, 0], degp[1, :, 0]], axis=1)
  d2 = degp[:, :, 0].reshape(NC, NPP, 2)
  degt2 = jnp.stack([d2[0, :, 0], d2[1, :, 0], d2[0, :, 1], d2[1, :, 1]],
                    axis=1)
  d8 = degp[:, :, 0].reshape(NC, NP // 8, 8)
  degt8 = jnp.stack([d8[0], d8[1]], axis=2).reshape(NP // 8, 16)

  h = _one_block(_tc_h_body, (NP, HID))(x, W1)
  g1 = _one_block(_tc_g1_body, (NP, HID))(h, degt)

  g2n = g1.reshape(2 * NP, HH)
  pp1 = _edge_pass_wide(g2n, srca, srcb, dstf, zeros64)
  ppv = pp1.reshape(NC, 2, NPP, 128)

  g2p = _one_block(_tc2_body, (NPP, 2 * CP16))(ppv, degt2, b1r, w2pp)

  g2view = g2p.reshape(NP, CP16)
  pp2 = _edge_pass_narrow(g2view, srcf, dstf, zeros16)
  qqv = pp2.reshape(NC, NP // 8, 128)

  outp = _one_block(_tc3_body, (NP // 8, 128))(qqv, degt8, b2q)
  return outp.reshape(NP, CP16)[:N, :C]


# speedup vs baseline: 1.1374x; 1.1217x over previous
import functools

import jax
import jax.numpy as jnp
from jax import lax
from jax.experimental import pallas as pl
from jax.experimental.pallas import tpu as pltpu
from jax.experimental.pallas import tpu_sc as plsc

N = 10000
E = 320000
F_IN = 128
HID = 128
C = 7
CP = 16
PK = 8

NC = 2
NS = 16
NW = NC * NS
EPW = E // NW
CHUNK = 80
NCHUNK = EPW // CHUNK
NP = 10240
RPT = NP // NS
NPQ = NP // PK
HH = HID // 2

_MESH = plsc.VectorSubcoreMesh(core_axis_name="c", subcore_axis_name="s")
_SC_PARAMS = pltpu.CompilerParams(use_tc_tiling_on_sc=False)

NBUF = 6
LOOK = 3
GRP = 25


def _edge_loop(g_hbm, src_v, dst_v, rows_v, acc_sh, gsem, ssem):
  def group(i, _):
    base = i * GRP
    gd, sd = {}, {}
    for k in range(LOOK):
      gd[k] = pltpu.async_copy(g_hbm.at[src_v.at[base + k]],
                               rows_v.at[k], gsem)
    for k in range(GRP):
      if k >= 2:
        sd[k - 2].wait()
      if k + LOOK < GRP:
        gd[k + LOOK] = pltpu.async_copy(g_hbm.at[src_v.at[base + k + LOOK]],
                                        rows_v.at[(k + LOOK) % NBUF], gsem)
      gd[k].wait()
      sd[k] = pltpu.async_copy(rows_v.at[k % NBUF],
                               acc_sh.at[dst_v.at[base + k]], ssem, add=True)
    sd[GRP - 2].wait()
    sd[GRP - 1].wait()
    return 0

  lax.fori_loop(0, NCHUNK // GRP, group, 0)


@functools.partial(
    pl.kernel,
    out_type=jax.ShapeDtypeStruct((NC, 2, NP, HH), jnp.float32),
    mesh=_MESH,
    compiler_params=_SC_PARAMS,
    scratch_types=[
        pltpu.VMEM((NCHUNK, CHUNK), jnp.int32),
        pltpu.VMEM((NCHUNK, CHUNK), jnp.int32),
        pltpu.VMEM((NCHUNK, CHUNK), jnp.int32),
        pltpu.VMEM((NBUF, CHUNK, HH), jnp.float32),
        pltpu.VMEM_SHARED((NP, HH), jnp.float32),
        pltpu.SemaphoreType.DMA,
        pltpu.SemaphoreType.DMA,
    ],
)
def _edge_pass_wide(g2n_hbm, srca_hbm, srcb_hbm, dst_hbm, zeros_hbm, out_hbm,
                    srca_v, srcb_v, dst_v, rows_v, acc_sh, gsem, ssem):
  cid = lax.axis_index("c")
  sid = lax.axis_index("s")
  wid = cid * NS + sid
  stripe = pl.ds(sid * RPT, RPT)
  pltpu.sync_copy(srca_hbm.at[wid], srca_v)
  pltpu.sync_copy(srcb_hbm.at[wid], srcb_v)
  pltpu.sync_copy(dst_hbm.at[wid], dst_v)
  pltpu.sync_copy(zeros_hbm.at[stripe], acc_sh.at[stripe])
  plsc.subcore_barrier()
  for h, src_v in enumerate((srca_v, srcb_v)):
    _edge_loop(g2n_hbm, src_v, dst_v, rows_v, acc_sh, gsem, ssem)
    plsc.subcore_barrier()
    pltpu.sync_copy(acc_sh.at[stripe], out_hbm.at[cid, h, stripe])
    if h == 0:
      pltpu.sync_copy(zeros_hbm.at[stripe], acc_sh.at[stripe])
      plsc.subcore_barrier()


@functools.partial(
    pl.kernel,
    out_type=jax.ShapeDtypeStruct((NC, NP, CP), jnp.float32),
    mesh=_MESH,
    compiler_params=_SC_PARAMS,
    scratch_types=[
        pltpu.VMEM((NCHUNK, CHUNK), jnp.int32),
        pltpu.VMEM((NCHUNK, CHUNK), jnp.int32),
        pltpu.VMEM((NBUF, CHUNK, CP), jnp.float32),
        pltpu.VMEM_SHARED((NP, CP), jnp.float32),
        pltpu.SemaphoreType.DMA,
        pltpu.SemaphoreType.DMA,
    ],
)
def _edge_pass_narrow(g_hbm, src_hbm, dst_hbm, zeros_hbm, out_hbm,
                      src_v, dst_v, rows_v, acc_sh, gsem, ssem):
  cid = lax.axis_index("c")
  sid = lax.axis_index("s")
  wid = cid * NS + sid
  stripe = pl.ds(sid * RPT, RPT)
  pltpu.sync_copy(src_hbm.at[wid], src_v)
  pltpu.sync_copy(dst_hbm.at[wid], dst_v)
  pltpu.sync_copy(zeros_hbm.at[stripe], acc_sh.at[stripe])
  plsc.subcore_barrier()
  _edge_loop(g_hbm, src_v, dst_v, rows_v, acc_sh, gsem, ssem)
  plsc.subcore_barrier()
  pltpu.sync_copy(acc_sh.at[stripe], out_hbm.at[cid, stripe])


@functools.partial(
    pl.kernel,
    out_type=jax.ShapeDtypeStruct((NC, NP, CP), jnp.float32),
    mesh=_MESH,
    compiler_params=_SC_PARAMS,
    scratch_types=[
        pltpu.VMEM((NCHUNK, CHUNK), jnp.int32),
        pltpu.VMEM((CHUNK, CP), jnp.float32),
        pltpu.VMEM_SHARED((NP, CP), jnp.float32),
        pltpu.SemaphoreType.DMA,
    ],
)
def _deg_pass(dst_hbm, ones_hbm, zeros_hbm, out_hbm,
              dst_v, ones_v, acc_sh, dsem):
  cid = lax.axis_index("c")
  sid = lax.axis_index("s")
  wid = cid * NS + sid
  stripe = pl.ds(sid * RPT, RPT)
  pltpu.sync_copy(zeros_hbm.at[stripe], acc_sh.at[stripe])
  pltpu.sync_copy(dst_hbm.at[wid], dst_v)
  pltpu.sync_copy(ones_hbm, ones_v)
  plsc.subcore_barrier()

  def group(i, _):
    sd = {}
    for k in range(GRP):
      if k >= 3:
        sd[k - 3].wait()
      sd[k] = pltpu.async_copy(ones_v, acc_sh.at[dst_v.at[i * GRP + k]],
                               dsem, add=True)
    for k in range(GRP - 3, GRP):
      sd[k].wait()
    return 0

  lax.fori_loop(0, NCHUNK // GRP, group, 0)
  plsc.subcore_barrier()
  pltpu.sync_copy(acc_sh.at[stripe], out_hbm.at[cid, stripe])


def _dis_k(dq_ref, k):
  deg = 1.0 + dq_ref[0, :, CP * k:CP * k + 1] + dq_ref[1, :, CP * k:CP * k + 1]
  return lax.rsqrt(deg)


def _tc_h_body(x_ref, w1_ref, h_ref):
  h_ref[0:N, :] = jnp.dot(x_ref[...], w1_ref[...],
                          preferred_element_type=jnp.float32)


def _tc_g1_body(hq_ref, dq_ref, g1q_ref):
  disq = jnp.concatenate(
      [jnp.broadcast_to(_dis_k(dq_ref, k), (NPQ, HID)) for k in range(PK)],
      axis=1)
  g1q_ref[...] = hq_ref[...] * disq


def _tc2_body(ppv_ref, g1q_ref, dq_ref, b1_ref, w2q_ref, g2q_ref):
  h1 = []
  for k in range(PK):
    psum = jnp.concatenate(
        [ppv_ref[0, 0, :, HH * k:HH * (k + 1)]
         + ppv_ref[1, 0, :, HH * k:HH * (k + 1)],
         ppv_ref[0, 1, :, HH * k:HH * (k + 1)]
         + ppv_ref[1, 1, :, HH * k:HH * (k + 1)]], axis=1)
    g1k = g1q_ref[:, HID * k:HID * (k + 1)]
    agg = _dis_k(dq_ref, k) * (psum + g1k) + b1_ref[...]
    h1.append(jnp.maximum(agg, 0.0))
  h1q = jnp.concatenate(h1, axis=1)
  h2q = jnp.dot(h1q, w2q_ref[...], preferred_element_type=jnp.float32)
  disrow = jnp.concatenate(
      [jnp.broadcast_to(_dis_k(dq_ref, k), (NPQ, CP)) for k in range(PK)],
      axis=1)
  g2q_ref[...] = h2q * disrow


def _tc3_body(qqv_ref, g2q_ref, dq_ref, b2_ref, outq_ref):
  disrow = jnp.concatenate(
      [jnp.broadcast_to(_dis_k(dq_ref, k), (NPQ, CP)) for k in range(PK)],
      axis=1)
  outq_ref[...] = disrow * (qqv_ref[0] + qqv_ref[1] + g2q_ref[...]) \
      + b2_ref[...]


def _one_block(body, out_shape):
  return pl.pallas_call(
      body,
      out_shape=jax.ShapeDtypeStruct(out_shape, jnp.float32),
  )


def kernel(x, edge_index, W1, b1, W2, b2):
  src = edge_index[0]
  srca = (2 * src).reshape(NW, NCHUNK, CHUNK)
  srcb = (2 * src + 1).reshape(NW, NCHUNK, CHUNK)
  srcp = src.reshape(NW, NCHUNK, CHUNK)
  dst = edge_index[1].reshape(NW, NCHUNK, CHUNK)
  zeros64 = jnp.zeros((NP, HH), jnp.float32)
  zeros16 = jnp.zeros((NP, CP), jnp.float32)
  ones16 = jnp.ones((CHUNK, CP), jnp.float32)
  w2p = jnp.pad(W2, ((0, 0), (0, CP - C)))
  w2q = jnp.kron(jnp.eye(PK, dtype=jnp.float32), w2p)
  b1r = b1.reshape(1, HID)
  b2q = jnp.tile(jnp.pad(b2, (0, CP - C)), PK).reshape(1, PK * CP)

  degp = _deg_pass(dst, ones16, zeros16)
  degq = degp.reshape(NC, NPQ, PK * CP)

  h = _one_block(_tc_h_body, (NP, HID))(x, W1)
  hq = h.reshape(NPQ, PK * HID)
  g1q = _one_block(_tc_g1_body, (NPQ, PK * HID))(hq, degq)

  g2n = g1q.reshape(2 * NP, HH)
  pp1 = _edge_pass_wide(g2n, srca, srcb, dst, zeros64)
  ppv = pp1.reshape(NC, 2, NPQ, PK * HH)

  g2q = _one_block(_tc2_body, (NPQ, PK * CP))(ppv, g1q, degq, b1r, w2q)

  g2view = g2q.reshape(NP, CP)
  pp2 = _edge_pass_narrow(g2view, srcp, dst, zeros16)
  qqv = pp2.reshape(NC, NPQ, PK * CP)

  outq = _one_block(_tc3_body, (NPQ, PK * CP))(qqv, g2q, degq, b2q)
  return outq.reshape(NP, CP)[:N, :C]


# speedup vs baseline: 1.1565x; 1.0168x over previous
import functools

import jax
import jax.numpy as jnp
from jax import lax
from jax.experimental import pallas as pl
from jax.experimental.pallas import tpu as pltpu
from jax.experimental.pallas import tpu_sc as plsc

N = 10000
E = 320000
F_IN = 128
HID = 128
C = 7
CP = 16
PK = 8

NC = 2
NS = 16
NW = NC * NS
EPW = E // NW
CHUNK = 80
NCHUNK = EPW // CHUNK
NP = 10240
RPT = NP // NS
NPQ = NP // PK
HH = HID // 2

_MESH = plsc.VectorSubcoreMesh(core_axis_name="c", subcore_axis_name="s")
_SC_PARAMS = pltpu.CompilerParams(use_tc_tiling_on_sc=False)

NBUF = 8
LOOK = 4
GRP = 25


def _edge_loop(g_hbm, src_v, dst_v, rows_v, acc_sh, gsem, ssem):
  def group(i, _):
    base = i * GRP
    gd, sd = {}, {}
    for k in range(LOOK):
      gd[k] = pltpu.async_copy(g_hbm.at[src_v.at[base + k]],
                               rows_v.at[k], gsem)
    for k in range(GRP):
      if k >= 3:
        sd[k - 3].wait()
      if k + LOOK < GRP:
        gd[k + LOOK] = pltpu.async_copy(g_hbm.at[src_v.at[base + k + LOOK]],
                                        rows_v.at[(k + LOOK) % NBUF], gsem)
      gd[k].wait()
      sd[k] = pltpu.async_copy(rows_v.at[k % NBUF],
                               acc_sh.at[dst_v.at[base + k]], ssem, add=True)
    sd[GRP - 3].wait()
    sd[GRP - 2].wait()
    sd[GRP - 1].wait()
    return 0

  lax.fori_loop(0, NCHUNK // GRP, group, 0)


@functools.partial(
    pl.kernel,
    out_type=jax.ShapeDtypeStruct((NC, 2, NP, HH), jnp.float32),
    mesh=_MESH,
    compiler_params=_SC_PARAMS,
    scratch_types=[
        pltpu.VMEM((NCHUNK, CHUNK), jnp.int32),
        pltpu.VMEM((NCHUNK, CHUNK), jnp.int32),
        pltpu.VMEM((NCHUNK, CHUNK), jnp.int32),
        pltpu.VMEM((NBUF, CHUNK, HH), jnp.float32),
        pltpu.VMEM_SHARED((NP, HH), jnp.float32),
        pltpu.SemaphoreType.DMA,
        pltpu.SemaphoreType.DMA,
    ],
)
def _edge_pass_wide(g2n_hbm, srca_hbm, srcb_hbm, dst_hbm, zeros_hbm, out_hbm,
                    srca_v, srcb_v, dst_v, rows_v, acc_sh, gsem, ssem):
  cid = lax.axis_index("c")
  sid = lax.axis_index("s")
  wid = cid * NS + sid
  stripe = pl.ds(sid * RPT, RPT)
  pltpu.sync_copy(srca_hbm.at[wid], srca_v)
  pltpu.sync_copy(srcb_hbm.at[wid], srcb_v)
  pltpu.sync_copy(dst_hbm.at[wid], dst_v)
  pltpu.sync_copy(zeros_hbm.at[stripe], acc_sh.at[stripe])
  plsc.subcore_barrier()
  for h, src_v in enumerate((srca_v, srcb_v)):
    _edge_loop(g2n_hbm, src_v, dst_v, rows_v, acc_sh, gsem, ssem)
    plsc.subcore_barrier()
    pltpu.sync_copy(acc_sh.at[stripe], out_hbm.at[cid, h, stripe])
    if h == 0:
      pltpu.sync_copy(zeros_hbm.at[stripe], acc_sh.at[stripe])
      plsc.subcore_barrier()


@functools.partial(
    pl.kernel,
    out_type=jax.ShapeDtypeStruct((NC, NP, CP), jnp.float32),
    mesh=_MESH,
    compiler_params=_SC_PARAMS,
    scratch_types=[
        pltpu.VMEM((NCHUNK, CHUNK), jnp.int32),
        pltpu.VMEM((NCHUNK, CHUNK), jnp.int32),
        pltpu.VMEM((NBUF, CHUNK, CP), jnp.float32),
        pltpu.VMEM_SHARED((NP, CP), jnp.float32),
        pltpu.SemaphoreType.DMA,
        pltpu.SemaphoreType.DMA,
    ],
)
def _edge_pass_narrow(g_hbm, src_hbm, dst_hbm, zeros_hbm, out_hbm,
                      src_v, dst_v, rows_v, acc_sh, gsem, ssem):
  cid = lax.axis_index("c")
  sid = lax.axis_index("s")
  wid = cid * NS + sid
  stripe = pl.ds(sid * RPT, RPT)
  pltpu.sync_copy(src_hbm.at[wid], src_v)
  pltpu.sync_copy(dst_hbm.at[wid], dst_v)
  pltpu.sync_copy(zeros_hbm.at[stripe], acc_sh.at[stripe])
  plsc.subcore_barrier()
  _edge_loop(g_hbm, src_v, dst_v, rows_v, acc_sh, gsem, ssem)
  plsc.subcore_barrier()
  pltpu.sync_copy(acc_sh.at[stripe], out_hbm.at[cid, stripe])


@functools.partial(
    pl.kernel,
    out_type=jax.ShapeDtypeStruct((NC, NP, CP), jnp.float32),
    mesh=_MESH,
    compiler_params=_SC_PARAMS,
    scratch_types=[
        pltpu.VMEM((NCHUNK, CHUNK), jnp.int32),
        pltpu.VMEM((CHUNK, CP), jnp.float32),
        pltpu.VMEM_SHARED((NP, CP), jnp.float32),
        pltpu.SemaphoreType.DMA,
    ],
)
def _deg_pass(dst_hbm, ones_hbm, zeros_hbm, out_hbm,
              dst_v, ones_v, acc_sh, dsem):
  cid = lax.axis_index("c")
  sid = lax.axis_index("s")
  wid = cid * NS + sid
  stripe = pl.ds(sid * RPT, RPT)
  pltpu.sync_copy(zeros_hbm.at[stripe], acc_sh.at[stripe])
  pltpu.sync_copy(dst_hbm.at[wid], dst_v)
  pltpu.sync_copy(ones_hbm, ones_v)
  plsc.subcore_barrier()

  def group(i, _):
    sd = {}
    for k in range(GRP):
      if k >= 3:
        sd[k - 3].wait()
      sd[k] = pltpu.async_copy(ones_v, acc_sh.at[dst_v.at[i * GRP + k]],
                               dsem, add=True)
    for k in range(GRP - 3, GRP):
      sd[k].wait()
    return 0

  lax.fori_loop(0, NCHUNK // GRP, group, 0)
  plsc.subcore_barrier()
  pltpu.sync_copy(acc_sh.at[stripe], out_hbm.at[cid, stripe])


def _dis_k(dq_ref, k):
  deg = 1.0 + dq_ref[0, :, CP * k:CP * k + 1] + dq_ref[1, :, CP * k:CP * k + 1]
  return lax.rsqrt(deg)


def _tc_h_body(x_ref, w1_ref, h_ref):
  h_ref[0:N, :] = jnp.dot(x_ref[...], w1_ref[...],
                          preferred_element_type=jnp.float32)


def _tc_g1_body(hq_ref, dq_ref, g1q_ref):
  disq = jnp.concatenate(
      [jnp.broadcast_to(_dis_k(dq_ref, k), (NPQ, HID)) for k in range(PK)],
      axis=1)
  g1q_ref[...] = hq_ref[...] * disq


def _tc2_body(ppv_ref, g1q_ref, dq_ref, b1_ref, w2q_ref, g2q_ref):
  h1 = []
  for k in range(PK):
    psum = jnp.concatenate(
        [ppv_ref[0, 0, :, HH * k:HH * (k + 1)]
         + ppv_ref[1, 0, :, HH * k:HH * (k + 1)],
         ppv_ref[0, 1, :, HH * k:HH * (k + 1)]
         + ppv_ref[1, 1, :, HH * k:HH * (k + 1)]], axis=1)
    g1k = g1q_ref[:, HID * k:HID * (k + 1)]
    agg = _dis_k(dq_ref, k) * (psum + g1k) + b1_ref[...]
    h1.append(jnp.maximum(agg, 0.0))
  h1q = jnp.concatenate(h1, axis=1)
  h2q = jnp.dot(h1q, w2q_ref[...], preferred_element_type=jnp.float32)
  disrow = jnp.concatenate(
      [jnp.broadcast_to(_dis_k(dq_ref, k), (NPQ, CP)) for k in range(PK)],
      axis=1)
  g2q_ref[...] = h2q * disrow


def _tc3_body(qqv_ref, g2q_ref, dq_ref, b2_ref, outq_ref):
  disrow = jnp.concatenate(
      [jnp.broadcast_to(_dis_k(dq_ref, k), (NPQ, CP)) for k in range(PK)],
      axis=1)
  outq_ref[...] = disrow * (qqv_ref[0] + qqv_ref[1] + g2q_ref[...]) \
      + b2_ref[...]


def _one_block(body, out_shape):
  return pl.pallas_call(
      body,
      out_shape=jax.ShapeDtypeStruct(out_shape, jnp.float32),
  )


def kernel(x, edge_index, W1, b1, W2, b2):
  src = edge_index[0]
  srca = (2 * src).reshape(NW, NCHUNK, CHUNK)
  srcb = (2 * src + 1).reshape(NW, NCHUNK, CHUNK)
  srcp = src.reshape(NW, NCHUNK, CHUNK)
  dst = edge_index[1].reshape(NW, NCHUNK, CHUNK)
  zeros64 = jnp.zeros((NP, HH), jnp.float32)
  zeros16 = jnp.zeros((NP, CP), jnp.float32)
  ones16 = jnp.ones((CHUNK, CP), jnp.float32)
  w2p = jnp.pad(W2, ((0, 0), (0, CP - C)))
  w2q = jnp.kron(jnp.eye(PK, dtype=jnp.float32), w2p)
  b1r = b1.reshape(1, HID)
  b2q = jnp.tile(jnp.pad(b2, (0, CP - C)), PK).reshape(1, PK * CP)

  degp = _deg_pass(dst, ones16, zeros16)
  degq = degp.reshape(NC, NPQ, PK * CP)

  h = _one_block(_tc_h_body, (NP, HID))(x, W1)
  hq = h.reshape(NPQ, PK * HID)
  g1q = _one_block(_tc_g1_body, (NPQ, PK * HID))(hq, degq)

  g2n = g1q.reshape(2 * NP, HH)
  pp1 = _edge_pass_wide(g2n, srca, srcb, dst, zeros64)
  ppv = pp1.reshape(NC, 2, NPQ, PK * HH)

  g2q = _one_block(_tc2_body, (NPQ, PK * CP))(ppv, g1q, degq, b1r, w2q)

  g2view = g2q.reshape(NP, CP)
  pp2 = _edge_pass_narrow(g2view, srcp, dst, zeros16)
  qqv = pp2.reshape(NC, NPQ, PK * CP)

  outq = _one_block(_tc3_body, (NPQ, PK * CP))(qqv, g2q, degq, b2q)
  return outq.reshape(NP, CP)[:N, :C]


# speedup vs baseline: 1.1748x; 1.0159x over previous
import functools

import jax
import jax.numpy as jnp
from jax import lax
from jax.experimental import pallas as pl
from jax.experimental.pallas import tpu as pltpu
from jax.experimental.pallas import tpu_sc as plsc

N = 10000
E = 320000
F_IN = 128
HID = 128
C = 7
CP = 16
PK = 8

NC = 2
NS = 16
NW = NC * NS
EPW = E // NW
CHUNK = 80
NCHUNK = EPW // CHUNK
NP = 10240
RPT = NP // NS
NPQ = NP // PK
HH = HID // 2

_MESH = plsc.VectorSubcoreMesh(core_axis_name="c", subcore_axis_name="s")
_SC_PARAMS = pltpu.CompilerParams(use_tc_tiling_on_sc=False)

NBUF = 8
LOOK = 5
GRP = 25


def _edge_loop(g_hbm, src_v, dst_v, rows_v, acc_sh, gsem, ssem):
  def group(i, _):
    base = i * GRP
    gd, sd = {}, {}
    for k in range(LOOK):
      gd[k] = pltpu.async_copy(g_hbm.at[src_v.at[base + k]],
                               rows_v.at[k], gsem)
    for k in range(GRP):
      if k >= 3:
        sd[k - 3].wait()
      if k + LOOK < GRP:
        gd[k + LOOK] = pltpu.async_copy(g_hbm.at[src_v.at[base + k + LOOK]],
                                        rows_v.at[(k + LOOK) % NBUF], gsem)
      gd[k].wait()
      sd[k] = pltpu.async_copy(rows_v.at[k % NBUF],
                               acc_sh.at[dst_v.at[base + k]], ssem, add=True)
    sd[GRP - 3].wait()
    sd[GRP - 2].wait()
    sd[GRP - 1].wait()
    return 0

  lax.fori_loop(0, NCHUNK // GRP, group, 0)


@functools.partial(
    pl.kernel,
    out_type=jax.ShapeDtypeStruct((NC, 2, NP, HH), jnp.float32),
    mesh=_MESH,
    compiler_params=_SC_PARAMS,
    scratch_types=[
        pltpu.VMEM((NCHUNK, CHUNK), jnp.int32),
        pltpu.VMEM((NCHUNK, CHUNK), jnp.int32),
        pltpu.VMEM((NCHUNK, CHUNK), jnp.int32),
        pltpu.VMEM((NBUF, CHUNK, HH), jnp.float32),
        pltpu.VMEM_SHARED((NP, HH), jnp.float32),
        pltpu.SemaphoreType.DMA,
        pltpu.SemaphoreType.DMA,
    ],
)
def _edge_pass_wide(g2n_hbm, srca_hbm, srcb_hbm, dst_hbm, zeros_hbm, out_hbm,
                    srca_v, srcb_v, dst_v, rows_v, acc_sh, gsem, ssem):
  cid = lax.axis_index("c")
  sid = lax.axis_index("s")
  wid = cid * NS + sid
  stripe = pl.ds(sid * RPT, RPT)
  pltpu.sync_copy(srca_hbm.at[wid], srca_v)
  pltpu.sync_copy(srcb_hbm.at[wid], srcb_v)
  pltpu.sync_copy(dst_hbm.at[wid], dst_v)
  pltpu.sync_copy(zeros_hbm.at[stripe], acc_sh.at[stripe])
  plsc.subcore_barrier()
  for h, src_v in enumerate((srca_v, srcb_v)):
    _edge_loop(g2n_hbm, src_v, dst_v, rows_v, acc_sh, gsem, ssem)
    plsc.subcore_barrier()
    pltpu.sync_copy(acc_sh.at[stripe], out_hbm.at[cid, h, stripe])
    if h == 0:
      pltpu.sync_copy(zeros_hbm.at[stripe], acc_sh.at[stripe])
      plsc.subcore_barrier()


@functools.partial(
    pl.kernel,
    out_type=jax.ShapeDtypeStruct((NC, NP, CP), jnp.float32),
    mesh=_MESH,
    compiler_params=_SC_PARAMS,
    scratch_types=[
        pltpu.VMEM((NCHUNK, CHUNK), jnp.int32),
        pltpu.VMEM((NCHUNK, CHUNK), jnp.int32),
        pltpu.VMEM((NBUF, CHUNK, CP), jnp.float32),
        pltpu.VMEM_SHARED((NP, CP), jnp.float32),
        pltpu.SemaphoreType.DMA,
        pltpu.SemaphoreType.DMA,
    ],
)
def _edge_pass_narrow(g_hbm, src_hbm, dst_hbm, zeros_hbm, out_hbm,
                      src_v, dst_v, rows_v, acc_sh, gsem, ssem):
  cid = lax.axis_index("c")
  sid = lax.axis_index("s")
  wid = cid * NS + sid
  stripe = pl.ds(sid * RPT, RPT)
  pltpu.sync_copy(src_hbm.at[wid], src_v)
  pltpu.sync_copy(dst_hbm.at[wid], dst_v)
  pltpu.sync_copy(zeros_hbm.at[stripe], acc_sh.at[stripe])
  plsc.subcore_barrier()
  _edge_loop(g_hbm, src_v, dst_v, rows_v, acc_sh, gsem, ssem)
  plsc.subcore_barrier()
  pltpu.sync_copy(acc_sh.at[stripe], out_hbm.at[cid, stripe])


@functools.partial(
    pl.kernel,
    out_type=jax.ShapeDtypeStruct((NC, NP, CP), jnp.float32),
    mesh=_MESH,
    compiler_params=_SC_PARAMS,
    scratch_types=[
        pltpu.VMEM((NCHUNK, CHUNK), jnp.int32),
        pltpu.VMEM((CHUNK, CP), jnp.float32),
        pltpu.VMEM_SHARED((NP, CP), jnp.float32),
        pltpu.SemaphoreType.DMA,
    ],
)
def _deg_pass(dst_hbm, ones_hbm, zeros_hbm, out_hbm,
              dst_v, ones_v, acc_sh, dsem):
  cid = lax.axis_index("c")
  sid = lax.axis_index("s")
  wid = cid * NS + sid
  stripe = pl.ds(sid * RPT, RPT)
  pltpu.sync_copy(zeros_hbm.at[stripe], acc_sh.at[stripe])
  pltpu.sync_copy(dst_hbm.at[wid], dst_v)
  pltpu.sync_copy(ones_hbm, ones_v)
  plsc.subcore_barrier()

  def group(i, _):
    sd = {}
    for k in range(GRP):
      if k >= 3:
        sd[k - 3].wait()
      sd[k] = pltpu.async_copy(ones_v, acc_sh.at[dst_v.at[i * GRP + k]],
                               dsem, add=True)
    for k in range(GRP - 3, GRP):
      sd[k].wait()
    return 0

  lax.fori_loop(0, NCHUNK // GRP, group, 0)
  plsc.subcore_barrier()
  pltpu.sync_copy(acc_sh.at[stripe], out_hbm.at[cid, stripe])


def _dis_k(dq_ref, k):
  deg = 1.0 + dq_ref[0, :, CP * k:CP * k + 1] + dq_ref[1, :, CP * k:CP * k + 1]
  return lax.rsqrt(deg)


def _tc_h_body(x_ref, w1_ref, h_ref):
  h_ref[0:N, :] = jnp.dot(x_ref[...], w1_ref[...],
                          preferred_element_type=jnp.float32)


def _tc_g1_body(hq_ref, dq_ref, g1q_ref):
  disq = jnp.concatenate(
      [jnp.broadcast_to(_dis_k(dq_ref, k), (NPQ, HID)) for k in range(PK)],
      axis=1)
  g1q_ref[...] = hq_ref[...] * disq


def _tc2_body(ppv_ref, g1q_ref, dq_ref, b1_ref, w2q_ref, g2q_ref):
  h1 = []
  for k in range(PK):
    psum = jnp.concatenate(
        [ppv_ref[0, 0, :, HH * k:HH * (k + 1)]
         + ppv_ref[1, 0, :, HH * k:HH * (k + 1)],
         ppv_ref[0, 1, :, HH * k:HH * (k + 1)]
         + ppv_ref[1, 1, :, HH * k:HH * (k + 1)]], axis=1)
    g1k = g1q_ref[:, HID * k:HID * (k + 1)]
    agg = _dis_k(dq_ref, k) * (psum + g1k) + b1_ref[...]
    h1.append(jnp.maximum(agg, 0.0))
  h1q = jnp.concatenate(h1, axis=1)
  h2q = jnp.dot(h1q, w2q_ref[...], preferred_element_type=jnp.float32)
  disrow = jnp.concatenate(
      [jnp.broadcast_to(_dis_k(dq_ref, k), (NPQ, CP)) for k in range(PK)],
      axis=1)
  g2q_ref[...] = h2q * disrow


def _tc3_body(qqv_ref, g2q_ref, dq_ref, b2_ref, outq_ref):
  disrow = jnp.concatenate(
      [jnp.broadcast_to(_dis_k(dq_ref, k), (NPQ, CP)) for k in range(PK)],
      axis=1)
  outq_ref[...] = disrow * (qqv_ref[0] + qqv_ref[1] + g2q_ref[...]) \
      + b2_ref[...]


def _one_block(body, out_shape):
  return pl.pallas_call(
      body,
      out_shape=jax.ShapeDtypeStruct(out_shape, jnp.float32),
  )


def kernel(x, edge_index, W1, b1, W2, b2):
  src = edge_index[0]
  srca = (2 * src).reshape(NW, NCHUNK, CHUNK)
  srcb = (2 * src + 1).reshape(NW, NCHUNK, CHUNK)
  srcp = src.reshape(NW, NCHUNK, CHUNK)
  dst = edge_index[1].reshape(NW, NCHUNK, CHUNK)
  zeros64 = jnp.zeros((NP, HH), jnp.float32)
  zeros16 = jnp.zeros((NP, CP), jnp.float32)
  ones16 = jnp.ones((CHUNK, CP), jnp.float32)
  w2p = jnp.pad(W2, ((0, 0), (0, CP - C)))
  w2q = jnp.kron(jnp.eye(PK, dtype=jnp.float32), w2p)
  b1r = b1.reshape(1, HID)
  b2q = jnp.tile(jnp.pad(b2, (0, CP - C)), PK).reshape(1, PK * CP)

  degp = _deg_pass(dst, ones16, zeros16)
  degq = degp.reshape(NC, NPQ, PK * CP)

  h = _one_block(_tc_h_body, (NP, HID))(x, W1)
  hq = h.reshape(NPQ, PK * HID)
  g1q = _one_block(_tc_g1_body, (NPQ, PK * HID))(hq, degq)

  g2n = g1q.reshape(2 * NP, HH)
  pp1 = _edge_pass_wide(g2n, srca, srcb, dst, zeros64)
  ppv = pp1.reshape(NC, 2, NPQ, PK * HH)

  g2q = _one_block(_tc2_body, (NPQ, PK * CP))(ppv, g1q, degq, b1r, w2q)

  g2view = g2q.reshape(NP, CP)
  pp2 = _edge_pass_narrow(g2view, srcp, dst, zeros16)
  qqv = pp2.reshape(NC, NPQ, PK * CP)

  outq = _one_block(_tc3_body, (NPQ, PK * CP))(qqv, g2q, degq, b2q)
  return outq.reshape(NP, CP)[:N, :C]


# speedup vs baseline: 1.1815x; 1.0057x over previous
import functools

import jax
import jax.numpy as jnp
from jax import lax
from jax.experimental import pallas as pl
from jax.experimental.pallas import tpu as pltpu
from jax.experimental.pallas import tpu_sc as plsc

N = 10000
E = 320000
F_IN = 128
HID = 128
C = 7
CP = 16
PK = 8

NC = 2
NS = 16
NW = NC * NS
EPW = E // NW
CHUNK = 80
NCHUNK = EPW // CHUNK
NP = 10240
RPT = NP // NS
NPQ = NP // PK
HH = HID // 2

_MESH = plsc.VectorSubcoreMesh(core_axis_name="c", subcore_axis_name="s")
_SC_PARAMS = pltpu.CompilerParams(use_tc_tiling_on_sc=False)

NBUF = 10
LOOK = 6
GRP = 25


def _edge_loop(g_hbm, src_v, dst_v, rows_v, acc_sh, gsem, ssem):
  def group(i, _):
    base = i * GRP
    gd, sd = {}, {}
    for k in range(LOOK):
      gd[k] = pltpu.async_copy(g_hbm.at[src_v.at[base + k]],
                               rows_v.at[k], gsem)
    for k in range(GRP):
      if k >= 3:
        sd[k - 3].wait()
      if k + LOOK < GRP:
        gd[k + LOOK] = pltpu.async_copy(g_hbm.at[src_v.at[base + k + LOOK]],
                                        rows_v.at[(k + LOOK) % NBUF], gsem)
      gd[k].wait()
      sd[k] = pltpu.async_copy(rows_v.at[k % NBUF],
                               acc_sh.at[dst_v.at[base + k]], ssem, add=True)
    sd[GRP - 3].wait()
    sd[GRP - 2].wait()
    sd[GRP - 1].wait()
    return 0

  lax.fori_loop(0, NCHUNK // GRP, group, 0)


@functools.partial(
    pl.kernel,
    out_type=jax.ShapeDtypeStruct((NC, 2, NP, HH), jnp.float32),
    mesh=_MESH,
    compiler_params=_SC_PARAMS,
    scratch_types=[
        pltpu.VMEM((NCHUNK, CHUNK), jnp.int32),
        pltpu.VMEM((NCHUNK, CHUNK), jnp.int32),
        pltpu.VMEM((NCHUNK, CHUNK), jnp.int32),
        pltpu.VMEM((NBUF, CHUNK, HH), jnp.float32),
        pltpu.VMEM_SHARED((NP, HH), jnp.float32),
        pltpu.SemaphoreType.DMA,
        pltpu.SemaphoreType.DMA,
    ],
)
def _edge_pass_wide(g2n_hbm, srca_hbm, srcb_hbm, dst_hbm, zeros_hbm, out_hbm,
                    srca_v, srcb_v, dst_v, rows_v, acc_sh, gsem, ssem):
  cid = lax.axis_index("c")
  sid = lax.axis_index("s")
  wid = cid * NS + sid
  stripe = pl.ds(sid * RPT, RPT)
  pltpu.sync_copy(srca_hbm.at[wid], srca_v)
  pltpu.sync_copy(srcb_hbm.at[wid], srcb_v)
  pltpu.sync_copy(dst_hbm.at[wid], dst_v)
  pltpu.sync_copy(zeros_hbm.at[stripe], acc_sh.at[stripe])
  plsc.subcore_barrier()
  for h, src_v in enumerate((srca_v, srcb_v)):
    _edge_loop(g2n_hbm, src_v, dst_v, rows_v, acc_sh, gsem, ssem)
    plsc.subcore_barrier()
    pltpu.sync_copy(acc_sh.at[stripe], out_hbm.at[cid, h, stripe])
    if h == 0:
      pltpu.sync_copy(zeros_hbm.at[stripe], acc_sh.at[stripe])
      plsc.subcore_barrier()


@functools.partial(
    pl.kernel,
    out_type=jax.ShapeDtypeStruct((NC, NP, CP), jnp.float32),
    mesh=_MESH,
    compiler_params=_SC_PARAMS,
    scratch_types=[
        pltpu.VMEM((NCHUNK, CHUNK), jnp.int32),
        pltpu.VMEM((NCHUNK, CHUNK), jnp.int32),
        pltpu.VMEM((NBUF, CHUNK, CP), jnp.float32),
        pltpu.VMEM_SHARED((NP, CP), jnp.float32),
        pltpu.SemaphoreType.DMA,
        pltpu.SemaphoreType.DMA,
    ],
)
def _edge_pass_narrow(g_hbm, src_hbm, dst_hbm, zeros_hbm, out_hbm,
                      src_v, dst_v, rows_v, acc_sh, gsem, ssem):
  cid = lax.axis_index("c")
  sid = lax.axis_index("s")
  wid = cid * NS + sid
  stripe = pl.ds(sid * RPT, RPT)
  pltpu.sync_copy(src_hbm.at[wid], src_v)
  pltpu.sync_copy(dst_hbm.at[wid], dst_v)
  pltpu.sync_copy(zeros_hbm.at[stripe], acc_sh.at[stripe])
  plsc.subcore_barrier()
  _edge_loop(g_hbm, src_v, dst_v, rows_v, acc_sh, gsem, ssem)
  plsc.subcore_barrier()
  pltpu.sync_copy(acc_sh.at[stripe], out_hbm.at[cid, stripe])


@functools.partial(
    pl.kernel,
    out_type=jax.ShapeDtypeStruct((NC, NP, CP), jnp.float32),
    mesh=_MESH,
    compiler_params=_SC_PARAMS,
    scratch_types=[
        pltpu.VMEM((NCHUNK, CHUNK), jnp.int32),
        pltpu.VMEM((CHUNK, CP), jnp.float32),
        pltpu.VMEM_SHARED((NP, CP), jnp.float32),
        pltpu.SemaphoreType.DMA,
    ],
)
def _deg_pass(dst_hbm, ones_hbm, zeros_hbm, out_hbm,
              dst_v, ones_v, acc_sh, dsem):
  cid = lax.axis_index("c")
  sid = lax.axis_index("s")
  wid = cid * NS + sid
  stripe = pl.ds(sid * RPT, RPT)
  pltpu.sync_copy(zeros_hbm.at[stripe], acc_sh.at[stripe])
  pltpu.sync_copy(dst_hbm.at[wid], dst_v)
  pltpu.sync_copy(ones_hbm, ones_v)
  plsc.subcore_barrier()

  def group(i, _):
    sd = {}
    for k in range(GRP):
      if k >= 3:
        sd[k - 3].wait()
      sd[k] = pltpu.async_copy(ones_v, acc_sh.at[dst_v.at[i * GRP + k]],
                               dsem, add=True)
    for k in range(GRP - 3, GRP):
      sd[k].wait()
    return 0

  lax.fori_loop(0, NCHUNK // GRP, group, 0)
  plsc.subcore_barrier()
  pltpu.sync_copy(acc_sh.at[stripe], out_hbm.at[cid, stripe])


def _dis_k(dq_ref, k):
  deg = 1.0 + dq_ref[0, :, CP * k:CP * k + 1] + dq_ref[1, :, CP * k:CP * k + 1]
  return lax.rsqrt(deg)


def _tc_h_body(x_ref, w1_ref, h_ref):
  h_ref[0:N, :] = jnp.dot(x_ref[...], w1_ref[...],
                          preferred_element_type=jnp.float32)


def _tc_g1_body(hq_ref, dq_ref, g1q_ref):
  disq = jnp.concatenate(
      [jnp.broadcast_to(_dis_k(dq_ref, k), (NPQ, HID)) for k in range(PK)],
      axis=1)
  g1q_ref[...] = hq_ref[...] * disq


def _tc2_body(ppv_ref, g1q_ref, dq_ref, b1_ref, w2q_ref, g2q_ref):
  h1 = []
  for k in range(PK):
    psum = jnp.concatenate(
        [ppv_ref[0, 0, :, HH * k:HH * (k + 1)]
         + ppv_ref[1, 0, :, HH * k:HH * (k + 1)],
         ppv_ref[0, 1, :, HH * k:HH * (k + 1)]
         + ppv_ref[1, 1, :, HH * k:HH * (k + 1)]], axis=1)
    g1k = g1q_ref[:, HID * k:HID * (k + 1)]
    agg = _dis_k(dq_ref, k) * (psum + g1k) + b1_ref[...]
    h1.append(jnp.maximum(agg, 0.0))
  h1q = jnp.concatenate(h1, axis=1)
  h2q = jnp.dot(h1q, w2q_ref[...], preferred_element_type=jnp.float32)
  disrow = jnp.concatenate(
      [jnp.broadcast_to(_dis_k(dq_ref, k), (NPQ, CP)) for k in range(PK)],
      axis=1)
  g2q_ref[...] = h2q * disrow


def _tc3_body(qqv_ref, g2q_ref, dq_ref, b2_ref, outq_ref):
  disrow = jnp.concatenate(
      [jnp.broadcast_to(_dis_k(dq_ref, k), (NPQ, CP)) for k in range(PK)],
      axis=1)
  outq_ref[...] = disrow * (qqv_ref[0] + qqv_ref[1] + g2q_ref[...]) \
      + b2_ref[...]


def _one_block(body, out_shape):
  return pl.pallas_call(
      body,
      out_shape=jax.ShapeDtypeStruct(out_shape, jnp.float32),
  )


def kernel(x, edge_index, W1, b1, W2, b2):
  src = edge_index[0]
  srca = (2 * src).reshape(NW, NCHUNK, CHUNK)
  srcb = (2 * src + 1).reshape(NW, NCHUNK, CHUNK)
  srcp = src.reshape(NW, NCHUNK, CHUNK)
  dst = edge_index[1].reshape(NW, NCHUNK, CHUNK)
  zeros64 = jnp.zeros((NP, HH), jnp.float32)
  zeros16 = jnp.zeros((NP, CP), jnp.float32)
  ones16 = jnp.ones((CHUNK, CP), jnp.float32)
  w2p = jnp.pad(W2, ((0, 0), (0, CP - C)))
  w2q = jnp.kron(jnp.eye(PK, dtype=jnp.float32), w2p)
  b1r = b1.reshape(1, HID)
  b2q = jnp.tile(jnp.pad(b2, (0, CP - C)), PK).reshape(1, PK * CP)

  degp = _deg_pass(dst, ones16, zeros16)
  degq = degp.reshape(NC, NPQ, PK * CP)

  h = _one_block(_tc_h_body, (NP, HID))(x, W1)
  hq = h.reshape(NPQ, PK * HID)
  g1q = _one_block(_tc_g1_body, (NPQ, PK * HID))(hq, degq)

  g2n = g1q.reshape(2 * NP, HH)
  pp1 = _edge_pass_wide(g2n, srca, srcb, dst, zeros64)
  ppv = pp1.reshape(NC, 2, NPQ, PK * HH)

  g2q = _one_block(_tc2_body, (NPQ, PK * CP))(ppv, g1q, degq, b1r, w2q)

  g2view = g2q.reshape(NP, CP)
  pp2 = _edge_pass_narrow(g2view, srcp, dst, zeros16)
  qqv = pp2.reshape(NC, NPQ, PK * CP)

  outq = _one_block(_tc3_body, (NPQ, PK * CP))(qqv, g2q, degq, b2q)
  return outq.reshape(NP, CP)[:N, :C]


# speedup vs baseline: 1.1820x; 1.0004x over previous
import functools

import jax
import jax.numpy as jnp
from jax import lax
from jax.experimental import pallas as pl
from jax.experimental.pallas import tpu as pltpu
from jax.experimental.pallas import tpu_sc as plsc

N = 10000
E = 320000
F_IN = 128
HID = 128
C = 7
CP = 16
PK = 8

NC = 2
NS = 16
NW = NC * NS
EPW = E // NW
CHUNK = 80
NCHUNK = EPW // CHUNK
NP = 10240
RPT = NP // NS
NPQ = NP // PK
HH = HID // 2

_MESH = plsc.VectorSubcoreMesh(core_axis_name="c", subcore_axis_name="s")
_SC_PARAMS = pltpu.CompilerParams(use_tc_tiling_on_sc=False)

NBUF = 10
LOOK = 6
GRP = 25


def _edge_loop(g_hbm, src_v, dst_v, rows_v, acc_sh, gsem, ssem):
  def group(i, _):
    base = i * GRP
    gd, sd = {}, {}
    for k in range(LOOK):
      gd[k] = pltpu.async_copy(g_hbm.at[src_v.at[base + k]],
                               rows_v.at[k], gsem)
    for k in range(GRP):
      if k >= 4:
        sd[k - 4].wait()
      if k + LOOK < GRP:
        gd[k + LOOK] = pltpu.async_copy(g_hbm.at[src_v.at[base + k + LOOK]],
                                        rows_v.at[(k + LOOK) % NBUF], gsem)
      gd[k].wait()
      sd[k] = pltpu.async_copy(rows_v.at[k % NBUF],
                               acc_sh.at[dst_v.at[base + k]], ssem, add=True)
    for k in range(GRP - 4, GRP):
      sd[k].wait()
    return 0

  lax.fori_loop(0, NCHUNK // GRP, group, 0)


@functools.partial(
    pl.kernel,
    out_type=jax.ShapeDtypeStruct((NC, 2, NP, HH), jnp.float32),
    mesh=_MESH,
    compiler_params=_SC_PARAMS,
    scratch_types=[
        pltpu.VMEM((NCHUNK, CHUNK), jnp.int32),
        pltpu.VMEM((NCHUNK, CHUNK), jnp.int32),
        pltpu.VMEM((NCHUNK, CHUNK), jnp.int32),
        pltpu.VMEM((NBUF, CHUNK, HH), jnp.float32),
        pltpu.VMEM_SHARED((NP, HH), jnp.float32),
        pltpu.SemaphoreType.DMA,
        pltpu.SemaphoreType.DMA,
    ],
)
def _edge_pass_wide(g2n_hbm, srca_hbm, srcb_hbm, dst_hbm, zeros_hbm, out_hbm,
                    srca_v, srcb_v, dst_v, rows_v, acc_sh, gsem, ssem):
  cid = lax.axis_index("c")
  sid = lax.axis_index("s")
  wid = cid * NS + sid
  stripe = pl.ds(sid * RPT, RPT)
  pltpu.sync_copy(srca_hbm.at[wid], srca_v)
  pltpu.sync_copy(srcb_hbm.at[wid], srcb_v)
  pltpu.sync_copy(dst_hbm.at[wid], dst_v)
  pltpu.sync_copy(zeros_hbm.at[stripe], acc_sh.at[stripe])
  plsc.subcore_barrier()
  for h, src_v in enumerate((srca_v, srcb_v)):
    _edge_loop(g2n_hbm, src_v, dst_v, rows_v, acc_sh, gsem, ssem)
    plsc.subcore_barrier()
    pltpu.sync_copy(acc_sh.at[stripe], out_hbm.at[cid, h, stripe])
    if h == 0:
      pltpu.sync_copy(zeros_hbm.at[stripe], acc_sh.at[stripe])
      plsc.subcore_barrier()


@functools.partial(
    pl.kernel,
    out_type=jax.ShapeDtypeStruct((NC, NP, CP), jnp.float32),
    mesh=_MESH,
    compiler_params=_SC_PARAMS,
    scratch_types=[
        pltpu.VMEM((NCHUNK, CHUNK), jnp.int32),
        pltpu.VMEM((NCHUNK, CHUNK), jnp.int32),
        pltpu.VMEM((NBUF, CHUNK, CP), jnp.float32),
        pltpu.VMEM_SHARED((NP, CP), jnp.float32),
        pltpu.SemaphoreType.DMA,
        pltpu.SemaphoreType.DMA,
    ],
)
def _edge_pass_narrow(g_hbm, src_hbm, dst_hbm, zeros_hbm, out_hbm,
                      src_v, dst_v, rows_v, acc_sh, gsem, ssem):
  cid = lax.axis_index("c")
  sid = lax.axis_index("s")
  wid = cid * NS + sid
  stripe = pl.ds(sid * RPT, RPT)
  pltpu.sync_copy(src_hbm.at[wid], src_v)
  pltpu.sync_copy(dst_hbm.at[wid], dst_v)
  pltpu.sync_copy(zeros_hbm.at[stripe], acc_sh.at[stripe])
  plsc.subcore_barrier()
  _edge_loop(g_hbm, src_v, dst_v, rows_v, acc_sh, gsem, ssem)
  plsc.subcore_barrier()
  pltpu.sync_copy(acc_sh.at[stripe], out_hbm.at[cid, stripe])


@functools.partial(
    pl.kernel,
    out_type=jax.ShapeDtypeStruct((NC, NP, CP), jnp.float32),
    mesh=_MESH,
    compiler_params=_SC_PARAMS,
    scratch_types=[
        pltpu.VMEM((NCHUNK, CHUNK), jnp.int32),
        pltpu.VMEM((CHUNK, CP), jnp.float32),
        pltpu.VMEM_SHARED((NP, CP), jnp.float32),
        pltpu.SemaphoreType.DMA,
    ],
)
def _deg_pass(dst_hbm, ones_hbm, zeros_hbm, out_hbm,
              dst_v, ones_v, acc_sh, dsem):
  cid = lax.axis_index("c")
  sid = lax.axis_index("s")
  wid = cid * NS + sid
  stripe = pl.ds(sid * RPT, RPT)
  pltpu.sync_copy(zeros_hbm.at[stripe], acc_sh.at[stripe])
  pltpu.sync_copy(dst_hbm.at[wid], dst_v)
  pltpu.sync_copy(ones_hbm, ones_v)
  plsc.subcore_barrier()

  def group(i, _):
    sd = {}
    for k in range(GRP):
      if k >= 3:
        sd[k - 3].wait()
      sd[k] = pltpu.async_copy(ones_v, acc_sh.at[dst_v.at[i * GRP + k]],
                               dsem, add=True)
    for k in range(GRP - 3, GRP):
      sd[k].wait()
    return 0

  lax.fori_loop(0, NCHUNK // GRP, group, 0)
  plsc.subcore_barrier()
  pltpu.sync_copy(acc_sh.at[stripe], out_hbm.at[cid, stripe])


def _dis_k(dq_ref, k):
  deg = 1.0 + dq_ref[0, :, CP * k:CP * k + 1] + dq_ref[1, :, CP * k:CP * k + 1]
  return lax.rsqrt(deg)


def _tc_h_body(x_ref, w1_ref, h_ref):
  h_ref[0:N, :] = jnp.dot(x_ref[...], w1_ref[...],
                          preferred_element_type=jnp.float32)


def _tc_g1_body(hq_ref, dq_ref, g1q_ref):
  disq = jnp.concatenate(
      [jnp.broadcast_to(_dis_k(dq_ref, k), (NPQ, HID)) for k in range(PK)],
      axis=1)
  g1q_ref[...] = hq_ref[...] * disq


def _tc2_body(ppv_ref, g1q_ref, dq_ref, b1_ref, w2q_ref, g2q_ref):
  h1 = []
  for k in range(PK):
    psum = jnp.concatenate(
        [ppv_ref[0, 0, :, HH * k:HH * (k + 1)]
         + ppv_ref[1, 0, :, HH * k:HH * (k + 1)],
         ppv_ref[0, 1, :, HH * k:HH * (k + 1)]
         + ppv_ref[1, 1, :, HH * k:HH * (k + 1)]], axis=1)
    g1k = g1q_ref[:, HID * k:HID * (k + 1)]
    agg = _dis_k(dq_ref, k) * (psum + g1k) + b1_ref[...]
    h1.append(jnp.maximum(agg, 0.0))
  h1q = jnp.concatenate(h1, axis=1)
  h2q = jnp.dot(h1q, w2q_ref[...], preferred_element_type=jnp.float32)
  disrow = jnp.concatenate(
      [jnp.broadcast_to(_dis_k(dq_ref, k), (NPQ, CP)) for k in range(PK)],
      axis=1)
  g2q_ref[...] = h2q * disrow


def _tc3_body(qqv_ref, g2q_ref, dq_ref, b2_ref, outq_ref):
  disrow = jnp.concatenate(
      [jnp.broadcast_to(_dis_k(dq_ref, k), (NPQ, CP)) for k in range(PK)],
      axis=1)
  outq_ref[...] = disrow * (qqv_ref[0] + qqv_ref[1] + g2q_ref[...]) \
      + b2_ref[...]


def _one_block(body, out_shape):
  return pl.pallas_call(
      body,
      out_shape=jax.ShapeDtypeStruct(out_shape, jnp.float32),
  )


def kernel(x, edge_index, W1, b1, W2, b2):
  src = edge_index[0]
  srca = (2 * src).reshape(NW, NCHUNK, CHUNK)
  srcb = (2 * src + 1).reshape(NW, NCHUNK, CHUNK)
  srcp = src.reshape(NW, NCHUNK, CHUNK)
  dst = edge_index[1].reshape(NW, NCHUNK, CHUNK)
  zeros64 = jnp.zeros((NP, HH), jnp.float32)
  zeros16 = jnp.zeros((NP, CP), jnp.float32)
  ones16 = jnp.ones((CHUNK, CP), jnp.float32)
  w2p = jnp.pad(W2, ((0, 0), (0, CP - C)))
  w2q = jnp.kron(jnp.eye(PK, dtype=jnp.float32), w2p)
  b1r = b1.reshape(1, HID)
  b2q = jnp.tile(jnp.pad(b2, (0, CP - C)), PK).reshape(1, PK * CP)

  degp = _deg_pass(dst, ones16, zeros16)
  degq = degp.reshape(NC, NPQ, PK * CP)

  h = _one_block(_tc_h_body, (NP, HID))(x, W1)
  hq = h.reshape(NPQ, PK * HID)
  g1q = _one_block(_tc_g1_body, (NPQ, PK * HID))(hq, degq)

  g2n = g1q.reshape(2 * NP, HH)
  pp1 = _edge_pass_wide(g2n, srca, srcb, dst, zeros64)
  ppv = pp1.reshape(NC, 2, NPQ, PK * HH)

  g2q = _one_block(_tc2_body, (NPQ, PK * CP))(ppv, g1q, degq, b1r, w2q)

  g2view = g2q.reshape(NP, CP)
  pp2 = _edge_pass_narrow(g2view, srcp, dst, zeros16)
  qqv = pp2.reshape(NC, NPQ, PK * CP)

  outq = _one_block(_tc3_body, (NPQ, PK * CP))(qqv, g2q, degq, b2q)
  return outq.reshape(NP, CP)[:N, :C]
